# baseline (speedup 1.0000x reference)
import functools

import numpy as np
import jax
import jax.numpy as jnp
from jax import lax
from jax.experimental import pallas as pl
from jax.experimental.pallas import tpu as pltpu
from jax.experimental.pallas import tpu_sc as plsc

N = 10000
E = 320000
D = 128
ALPHA = 0.1
BETA = float(np.log(2.0))

NC = 2
NS = 16
NW = NC * NS
CH = 128
EP_T = 10240
NCH = EP_T // CH
EP = NW * EP_T
NP = 10240
STR = NP // NS
DUMMY = NP - 1

_MESH = plsc.VectorSubcoreMesh(core_axis_name="c", subcore_axis_name="s")


def _deg_body(col_hbm, zeros1_hbm, degp_hbm, col_v, ones_v, deg_sh):
    cid = lax.axis_index("c")
    sid = lax.axis_index("s")
    wid = cid * NS + sid
    pltpu.sync_copy(col_hbm.at[wid], col_v)
    ones16 = jnp.ones((16,), jnp.float32)
    for r in range(CH // 16):
        ones_v[pl.ds(r * 16, 16)] = ones16
    pltpu.sync_copy(zeros1_hbm.at[pl.ds(sid * STR, STR)],
                    deg_sh.at[pl.ds(sid * STR, STR)])
    plsc.subcore_barrier()

    def chunk(i, c):
        pltpu.sync_copy(ones_v, deg_sh.at[col_v.at[i]], add=True)
        return c

    lax.fori_loop(0, NCH, chunk, 0)
    plsc.subcore_barrier()
    pltpu.sync_copy(deg_sh.at[pl.ds(sid * STR, STR)],
                    degp_hbm.at[pl.ds(cid * NP + sid * STR, STR)])


_deg_call = pl.kernel(
    _deg_body,
    mesh=_MESH,
    out_type=jax.ShapeDtypeStruct((NC * NP,), jnp.float32),
    scratch_types=[
        pltpu.VMEM((NCH, CH), jnp.int32),
        pltpu.VMEM((CH,), jnp.float32),
        pltpu.VMEM_SHARED((NP,), jnp.float32),
    ],
)


TOT_CH = NW * NCH
T_T = TOT_CH // NS
HALF = T_T // 2


def _gs_body(y_hbm, combo_hbm, p_hbm, combo_v, row_st, col_st,
             b0, b1, accum, sg0, sg1, ss0, ss1):
    bufs = [b0, b1]
    sgs = [sg0, sg1]
    sss = [ss0, ss1]
    cid = lax.axis_index("c")
    sid = lax.axis_index("s")

    @pl.when(cid == 0)
    def _():
        pltpu.sync_copy(combo_hbm.at[pl.ds(sid * T_T, HALF)], combo_v)

        zeros16 = jnp.zeros((16,), jnp.float32)

        def zrow(r, c):
            for k in range(D // 16):
                b0[r, pl.ds(k * 16, 16)] = zeros16
            return c

        lax.fori_loop(0, CH, zrow, 0)
        for t in range(STR // CH):
            pltpu.sync_copy(b0, accum.at[pl.ds(sid * STR + t * CH, CH)])
        plsc.subcore_barrier()

        def decode(j, s):
            for k in range(CH // 16):
                v = combo_v[j, pl.ds(k * 16, 16)]
                row_st[s, pl.ds(k * 16, 16)] = lax.shift_right_logical(v, 14)
                col_st[s, pl.ds(k * 16, 16)] = lax.bitwise_and(v, 16383)

        decode(0, 0)
        pltpu.async_copy(y_hbm.at[row_st.at[0]], bufs[0], sgs[0])

        def body(k2, c):
            for b in range(2):
                j = 2 * k2 + b
                nb = 1 - b
                pltpu.make_async_copy(y_hbm.at[row_st.at[b]], bufs[b],
                                      sgs[b]).wait()
                pltpu.async_copy(bufs[b], accum.at[col_st.at[b]], sss[b],
                                 add=True)

                @pl.when(j + 1 < T_T)
                def _():
                    @pl.when(j + 1 == HALF)
                    def _():
                        pltpu.sync_copy(
                            combo_hbm.at[pl.ds(sid * T_T + HALF, HALF)],
                            combo_v)

                    dj = lax.select(j + 1 >= HALF, j + 1 - HALF, j + 1)
                    decode(dj, nb)

                    @pl.when(j >= 1)
                    def _():
                        pltpu.make_async_copy(bufs[nb],
                                              accum.at[col_st.at[nb]],
                                              sss[nb]).wait()

                    pltpu.async_copy(y_hbm.at[row_st.at[nb]], bufs[nb],
                                     sgs[nb])
            return c

        lax.fori_loop(0, T_T // 2, body, 0)
        for b in range(2):
            pltpu.make_async_copy(bufs[b], accum.at[col_st.at[b]],
                                  sss[b]).wait()
        plsc.subcore_barrier()
        pltpu.sync_copy(accum.at[pl.ds(sid * STR, STR)],
                        p_hbm.at[pl.ds(sid * STR, STR)])


_gs_call = pl.kernel(
    _gs_body,
    mesh=_MESH,
    out_type=jax.ShapeDtypeStruct((NP, D), jnp.float32),
    scratch_types=[
        pltpu.VMEM((HALF, CH), jnp.int32),
        pltpu.VMEM((2, CH), jnp.int32),
        pltpu.VMEM((2, CH), jnp.int32),
        pltpu.VMEM((CH, D), jnp.float32),
        pltpu.VMEM((CH, D), jnp.float32),
        pltpu.VMEM_SHARED((NP, D), jnp.float32),
        pltpu.SemaphoreType.DMA,
        pltpu.SemaphoreType.DMA,
        pltpu.SemaphoreType.DMA,
        pltpu.SemaphoreType.DMA,
    ],
)


def _y_body(degp_ref, x_ref, y_ref):
    deg = jnp.sum(degp_ref[...], axis=0) + 1.0
    dinv = lax.rsqrt(deg)
    y_ref[...] = x_ref[...] * dinv


_y_call = pl.pallas_call(
    _y_body, out_shape=jax.ShapeDtypeStruct((N, D), jnp.float32))


def _comb_body(degp_ref, p_ref, x_ref, x0_ref, w_ref, o_ref):
    deg = jnp.sum(degp_ref[...], axis=0) + 1.0
    dinv = lax.rsqrt(deg)
    s = p_ref[...][:N, :]
    x = x_ref[...]
    ax = dinv * s + (dinv * dinv) * x
    h = (1.0 - ALPHA) * ax + ALPHA * x0_ref[...]
    hw = jnp.dot(h, w_ref[...], preferred_element_type=jnp.float32)
    out = (1.0 - BETA) * h + BETA * hw
    o_ref[...] = x + jnp.maximum(out, 0.0)


_comb_call = pl.pallas_call(
    _comb_body, out_shape=jax.ShapeDtypeStruct((N, D), jnp.float32))


def kernel(x, x0, edge_index, W1):
    row = edge_index[0]
    col = edge_index[1]
    pad = jnp.full((EP - E,), DUMMY, jnp.int32)
    rowp = jnp.concatenate([row, pad]).reshape(NW, NCH, CH)
    colp = jnp.concatenate([col, pad]).reshape(NW, NCH, CH)
    zeros1 = jnp.zeros((NP,), jnp.float32)
    degp = _deg_call(colp, zeros1)
    degn = degp.reshape(NC, NP)[:, :N, None]
    y = _y_call(degn, x)
    ypad = jnp.concatenate([y, jnp.zeros((NP - N, D), jnp.float32)], axis=0)
    combo = (rowp * 16384 + colp).reshape(TOT_CH, CH)
    p = _gs_call(ypad, combo)
    return _comb_call(degn, p, x, x0, W1)


# speedup vs baseline: 1.3792x; 1.3792x over previous
import functools

import numpy as np
import jax
import jax.numpy as jnp
from jax import lax
from jax.experimental import pallas as pl
from jax.experimental.pallas import tpu as pltpu
from jax.experimental.pallas import tpu_sc as plsc

N = 10000
E = 320000
D = 128
ALPHA = 0.1
BETA = float(np.log(2.0))

NC = 2
NS = 16
NW = NC * NS
CH = 128
EP_T = 10240
NCH = EP_T // CH
EP = NW * EP_T
NP = 10240
STR = NP // NS
DUMMY = NP - 1

_MESH = plsc.VectorSubcoreMesh(core_axis_name="c", subcore_axis_name="s")


def _deg_body(col_hbm, zeros1_hbm, degp_hbm, col_v, ones_v, deg_sh):
    cid = lax.axis_index("c")
    sid = lax.axis_index("s")
    wid = cid * NS + sid
    pltpu.sync_copy(col_hbm.at[wid], col_v)
    ones16 = jnp.ones((16,), jnp.float32)
    for r in range(CH // 16):
        ones_v[pl.ds(r * 16, 16)] = ones16
    pltpu.sync_copy(zeros1_hbm.at[pl.ds(sid * STR, STR)],
                    deg_sh.at[pl.ds(sid * STR, STR)])
    plsc.subcore_barrier()

    def chunk(i, c):
        pltpu.sync_copy(ones_v, deg_sh.at[col_v.at[i]], add=True)
        return c

    lax.fori_loop(0, NCH, chunk, 0)
    plsc.subcore_barrier()
    pltpu.sync_copy(deg_sh.at[pl.ds(sid * STR, STR)],
                    degp_hbm.at[pl.ds(cid * NP + sid * STR, STR)])


_deg_call = pl.kernel(
    _deg_body,
    mesh=_MESH,
    out_type=jax.ShapeDtypeStruct((NC * NP,), jnp.float32),
    scratch_types=[
        pltpu.VMEM((NCH, CH), jnp.int32),
        pltpu.VMEM((CH,), jnp.float32),
        pltpu.VMEM_SHARED((NP,), jnp.float32),
    ],
)


TOT_CH = NW * NCH
F_T = 120
S_T = 40
F_BASE = NS * F_T


def _gs_body(y_hbm, combo_hbm, p_hbm, combo_v, row_st, col_st,
             b0, b1, accum, sg0, sg1, ss0, ss1):
    bufs = [b0, b1]
    sgs = [sg0, sg1]
    sss = [ss0, ss1]
    cid = lax.axis_index("c")
    sid = lax.axis_index("s")

    @pl.when(cid == 0)
    def _():
        pltpu.sync_copy(combo_hbm.at[pl.ds(sid * F_T, F_T)], combo_v)

    @pl.when(cid == 1)
    def _():
        pltpu.sync_copy(combo_hbm.at[pl.ds(F_BASE + sid * S_T, S_T)],
                        combo_v.at[pl.ds(0, S_T)])

    n = lax.select(cid == 0, F_T, S_T)

    zeros16 = jnp.zeros((16,), jnp.float32)

    def zrow(r, c):
        for k in range(D // 16):
            b0[r, pl.ds(k * 16, 16)] = zeros16
        return c

    lax.fori_loop(0, CH, zrow, 0)
    for t in range(STR // CH):
        pltpu.sync_copy(b0, accum.at[pl.ds(sid * STR + t * CH, CH)])
    plsc.subcore_barrier()

    def decode(j, s):
        for k in range(CH // 16):
            v = combo_v[j, pl.ds(k * 16, 16)]
            row_st[s, pl.ds(k * 16, 16)] = lax.shift_right_logical(v, 14)
            col_st[s, pl.ds(k * 16, 16)] = lax.bitwise_and(v, 16383)

    decode(0, 0)
    pltpu.async_copy(y_hbm.at[row_st.at[0]], bufs[0], sgs[0])

    def body(k2, c):
        for b in range(2):
            j = 2 * k2 + b
            nb = 1 - b
            pltpu.make_async_copy(y_hbm.at[row_st.at[b]], bufs[b],
                                  sgs[b]).wait()
            pltpu.async_copy(bufs[b], accum.at[col_st.at[b]], sss[b],
                             add=True)

            @pl.when(j + 1 < n)
            def _():
                decode(j + 1, nb)

                @pl.when(j >= 1)
                def _():
                    pltpu.make_async_copy(bufs[nb], accum.at[col_st.at[nb]],
                                          sss[nb]).wait()

                pltpu.async_copy(y_hbm.at[row_st.at[nb]], bufs[nb], sgs[nb])
        return c

    lax.fori_loop(0, lax.select(cid == 0, F_T // 2, S_T // 2), body, 0)
    for b in range(2):
        pltpu.make_async_copy(bufs[b], accum.at[col_st.at[b]], sss[b]).wait()
    plsc.subcore_barrier()
    pltpu.sync_copy(accum.at[pl.ds(sid * STR, STR)],
                    p_hbm.at[cid, pl.ds(sid * STR, STR)])


_gs_call = pl.kernel(
    _gs_body,
    mesh=_MESH,
    out_type=jax.ShapeDtypeStruct((NC, NP, D), jnp.float32),
    scratch_types=[
        pltpu.VMEM((F_T, CH), jnp.int32),
        pltpu.VMEM((2, CH), jnp.int32),
        pltpu.VMEM((2, CH), jnp.int32),
        pltpu.VMEM((CH, D), jnp.float32),
        pltpu.VMEM((CH, D), jnp.float32),
        pltpu.VMEM_SHARED((NP, D), jnp.float32),
        pltpu.SemaphoreType.DMA,
        pltpu.SemaphoreType.DMA,
        pltpu.SemaphoreType.DMA,
        pltpu.SemaphoreType.DMA,
    ],
)


def _y_body(degp_ref, x_ref, y_ref):
    deg = jnp.sum(degp_ref[...], axis=0) + 1.0
    dinv = lax.rsqrt(deg)
    y_ref[...] = x_ref[...] * dinv


_y_call = pl.pallas_call(
    _y_body, out_shape=jax.ShapeDtypeStruct((N, D), jnp.float32))


def _comb_body(degp_ref, p_ref, x_ref, x0_ref, w_ref, o_ref):
    deg = jnp.sum(degp_ref[...], axis=0) + 1.0
    dinv = lax.rsqrt(deg)
    p = p_ref[...]
    s = p[0, :N, :] + p[1, :N, :]
    x = x_ref[...]
    ax = dinv * s + (dinv * dinv) * x
    h = (1.0 - ALPHA) * ax + ALPHA * x0_ref[...]
    hw = jnp.dot(h, w_ref[...], preferred_element_type=jnp.float32)
    out = (1.0 - BETA) * h + BETA * hw
    o_ref[...] = x + jnp.maximum(out, 0.0)


_comb_call = pl.pallas_call(
    _comb_body, out_shape=jax.ShapeDtypeStruct((N, D), jnp.float32))


def kernel(x, x0, edge_index, W1):
    row = edge_index[0]
    col = edge_index[1]
    pad = jnp.full((EP - E,), DUMMY, jnp.int32)
    rowp = jnp.concatenate([row, pad]).reshape(NW, NCH, CH)
    colp = jnp.concatenate([col, pad]).reshape(NW, NCH, CH)
    zeros1 = jnp.zeros((NP,), jnp.float32)
    degp = _deg_call(colp, zeros1)
    degn = degp.reshape(NC, NP)[:, :N, None]
    y = _y_call(degn, x)
    ypad = jnp.concatenate([y, jnp.zeros((NP - N, D), jnp.float32)], axis=0)
    combo = (rowp * 16384 + colp).reshape(TOT_CH, CH)
    p = _gs_call(ypad, combo)
    return _comb_call(degn, p, x, x0, W1)
